# Initial kernel scaffold; baseline (speedup 1.0000x reference)
import jax
import jax.numpy as jnp
from jax.experimental import pallas as pl


def kernel(wind_direction, wind_speed, yaw, layout, node_in_w, node_in_b, edge_in_w, edge_in_b, msg_w1, msg_b1, msg_w2, msg_b2, upd_w1, upd_b1, upd_w2, upd_b2, att_w, att_b):
    raise NotImplementedError("write your pallas kernel here")


# speedup vs baseline: 5.7382x; 5.7382x over previous
import functools

import jax
import jax.numpy as jnp
from jax.experimental import pallas as pl
from jax.experimental.pallas import tpu as pltpu

_N = 100
_NI = 104
_NS = 16
_MAP_X = 5000.0
_MAP_Y = 5000.0
_WS_SCALE = 1.0 / 28.0
_DEG = 3.141592653589793 / 180.0
_L = 4


def _silu(x):
    return x * jax.nn.sigmoid(x)


def _fwd(cols_ref, layj_ref, niw_ref, nib_ref, ewt_ref, ebt_ref,
         wat_ref, wbm_ref, k1_ref, b1_ref, k2_ref, b2_ref,
         katt_ref, attb_ref, uw1h_ref, uw1a_ref, ub1_ref, uw2_ref, ub2_ref,
         sum8_ref, sall_ref, mask_ref, out_ref):
    cols = cols_ref[0]
    layj = layj_ref[0]
    mask3 = mask_ref[...]
    sum8 = sum8_ref[...]

    ws = cols[:, 0:1] * _WS_SCALE
    wd = cols[:, 1:2] * _DEG
    ywr = cols[:, 2:3] * _DEG
    lxi = cols[:, 3:4] * (2.0 / _MAP_X) - 1.0
    lyi = cols[:, 4:5] * (2.0 / _MAP_Y) - 1.0
    wx = ws * jnp.cos(wd)
    wy = ws * jnp.sin(wd)

    lxj = layj[0] * (2.0 / _MAP_X) - 1.0
    lyj = layj[1] * (2.0 / _MAP_Y) - 1.0

    lxi_b = lxi[:, :, None]
    lyi_b = lyi[:, :, None]
    wx_b = wx[:, :, None]
    wy_b = wy[:, :, None]
    X = lxj[None, :, :] - lxi_b
    Y = lyj[None, :, :] - lyi_b
    radial = jnp.sqrt(X * X + Y * Y)
    wdot = wx_b * X + wy_b * Y
    wcross = wx_b * Y - wy_b * X

    ewt = ewt_ref[...]
    ebt = ebt_ref[...]
    ws_b = ws[:, :, None]
    yw_b = ywr[:, :, None]
    e = (radial * ewt[0][None, None, :]
         + ws_b * ewt[1][None, None, :]
         + wdot * ewt[2][None, None, :]
         + wcross * ewt[3][None, None, :]
         + yw_b * ewt[4][None, None, :]
         + ebt[0][None, None, :])

    def agg(f):
        fm = f * mask3
        rs = jnp.sum(fm, axis=1)
        return jnp.dot(rs, sum8)[:, 0:1]

    a_rad = agg(radial)
    a_dot = agg(wdot)
    a_cross = agg(wcross)
    n_valid = 99.0
    niw = niw_ref[...]
    h = (ws * niw[0:1]
         + a_rad * niw[1:2]
         + (ws * n_valid) * niw[2:3]
         + a_dot * niw[3:4]
         + a_cross * niw[4:5]
         + (ywr * n_valid) * niw[5:6]
         + nib_ref[...])

    attb = attb_ref[0, 0]
    for li in range(_L):
        a_row = jnp.dot(h, wat_ref[li])
        bstack = jnp.dot(jnp.dot(sall_ref[...], h), wbm_ref[li])
        bp = jnp.concatenate(
            [bstack[16 * e:16 * (e + 1), :] for e in range(8)], axis=1)
        base = jnp.dot(e.reshape(_NI * _NS, 128), k1_ref[li]) + b1_ref[li]
        t = base.reshape(_NI, _NS, 128) + a_row[:, None, :] + bp[None, :, :]
        t = _silu(t)
        msg = _silu(jnp.dot(t.reshape(_NI * _NS, 128), k2_ref[li])
                    + b2_ref[li])
        att = jax.nn.sigmoid(jnp.dot(msg, katt_ref[...]) + attb)
        msg = msg * att
        m3 = msg.reshape(_NI, _NS, 128) * mask3
        h_aggr = jnp.dot(jnp.sum(m3, axis=1), sum8)
        u = _silu(jnp.dot(h, uw1h_ref[li]) + jnp.dot(h_aggr, uw1a_ref[li])
                  + ub1_ref[li])
        h_upd = _silu(jnp.dot(u, uw2_ref[li]) + ub2_ref[li])
        h = h + h_upd
        if li < _L - 1:
            e = e + msg.reshape(_NI, _NS, 128)

    out_ref[0] = h


@functools.partial(jax.jit, static_argnames=("interpret",))
def _run(wind_direction, wind_speed, yaw, layout, node_in_w, node_in_b,
         edge_in_w, edge_in_b, msg_w1, msg_b1, msg_w2, msg_b2,
         upd_w1, upd_b1, upd_w2, upd_b2, att_w, att_b, interpret=False):
    f32 = jnp.float32
    B = wind_direction.shape[0]

    cols = jnp.concatenate([wind_speed, wind_direction, yaw, layout], axis=2)
    cols = jnp.pad(cols, ((0, 0), (0, _NI - _N), (0, 0)))

    layp = jnp.pad(layout, ((0, 0), (0, 128 - _N), (0, 0)))
    layj = jnp.repeat(jnp.swapaxes(layp, 1, 2), 16, axis=2)
    layj = layj.reshape(B, 2, _NS, 128)

    i8 = jnp.eye(8, dtype=f32)
    kron8 = jax.vmap(lambda w: jnp.kron(i8, w))
    wat = jnp.tile(msg_w1[:, :64, :], (1, 1, 8))
    wbm = msg_w1[:, 64:128, :]
    k1 = kron8(msg_w1[:, 128:144, :])
    b1t = jnp.tile(msg_b1, (1, 8))[:, None, :]
    k2 = kron8(msg_w2)
    b2t = jnp.tile(msg_b2, (1, 8))[:, None, :]
    katt = jnp.kron(i8, att_w @ jnp.ones((1, 16), f32))
    attb = att_b.reshape(1, 1)
    ewt = jnp.tile(edge_in_w, (1, 8))
    ebt = jnp.tile(edge_in_b, (8,))[None, :]
    sum8 = jnp.tile(jnp.eye(16, dtype=f32), (8, 1))
    nib = node_in_b[None, :]
    uw1h = upd_w1[:, :64, :]
    uw2 = upd_w2
    uw1a = upd_w1[:, 64:, :]
    ub1 = upd_b1[:, None, :]
    ub2 = upd_b2[:, None, :]

    rr = jnp.arange(128)
    jsel = 8 * (rr % 16) + rr // 16
    sall = (jsel[:, None] == jnp.arange(_NI)[None, :]).astype(f32)

    ii = jnp.arange(_NI)[:, None, None]
    ss = jnp.arange(_NS)[None, :, None]
    qq = jnp.arange(128)[None, None, :]
    jj = 8 * ss + qq // 16
    mask3 = ((jj != ii) & (jj < _N)).astype(f32)

    def cmap(*shape):
        return pl.BlockSpec(shape, lambda b: (0,) * len(shape))

    in_specs = [
        pl.BlockSpec((1, _NI, 5), lambda b: (b, 0, 0)),
        pl.BlockSpec((1, 2, _NS, 128), lambda b: (b, 0, 0, 0)),
        cmap(6, 64), cmap(1, 64), cmap(5, 128), cmap(1, 128),
        cmap(_L, 64, 128), cmap(_L, 64, 16), cmap(_L, 128, 128),
        cmap(_L, 1, 128), cmap(_L, 128, 128), cmap(_L, 1, 128),
        cmap(128, 128), cmap(1, 1),
        cmap(_L, 64, 64), cmap(_L, 16, 64), cmap(_L, 1, 64),
        cmap(_L, 64, 64), cmap(_L, 1, 64),
        cmap(128, 16), cmap(128, _NI), cmap(_NI, _NS, 128),
    ]

    out = pl.pallas_call(
        _fwd,
        grid=(B,),
        in_specs=in_specs,
        out_specs=pl.BlockSpec((1, _NI, 64), lambda b: (b, 0, 0)),
        out_shape=jax.ShapeDtypeStruct((B, _NI, 64), f32),
        compiler_params=pltpu.CompilerParams(
            dimension_semantics=("arbitrary",)),
        interpret=interpret,
    )(cols, layj, node_in_w, nib, ewt, ebt, wat, wbm, k1, b1t, k2, b2t,
      katt, attb, uw1h, uw1a, ub1, uw2, ub2, sum8, sall, mask3)
    return out[:, :_N, :]


def kernel(wind_direction, wind_speed, yaw, layout, node_in_w, node_in_b,
           edge_in_w, edge_in_b, msg_w1, msg_b1, msg_w2, msg_b2,
           upd_w1, upd_b1, upd_w2, upd_b2, att_w, att_b):
    return _run(wind_direction, wind_speed, yaw, layout, node_in_w, node_in_b,
                edge_in_w, edge_in_b, msg_w1, msg_b1, msg_w2, msg_b2,
                upd_w1, upd_b1, upd_w2, upd_b2, att_w, att_b)
